# baseline (speedup 1.0000x reference)
import numpy as np
import jax
import jax.numpy as jnp
from jax.experimental import pallas as pl
from jax.experimental.pallas import tpu as pltpu

_R = 5
_NEG = -1e9
_TH = 128
_HALO = _R + 11

_D2S = sorted({dy * dy + dx * dx
               for dy in range(-_R, _R + 1) for dx in range(-_R, _R + 1)
               if dy * dy + dx * dx < (_R + 0.5) ** 2})
_KOF = {d2: k for k, d2 in enumerate(_D2S)}
_YMAX = [int(np.floor(np.sqrt((_R + 0.5) ** 2 - 1e-9 - dx * dx)))
         for dx in range(_R + 1)]


def _bokeh_body(img_ref, d_ref, bokeh_ref, dd_ref, img_scr, wpl_ref, cand_ref,
                img_sh_ref, b_ref, wc_ref):
    TH = bokeh_ref.shape[2]
    W = bokeh_ref.shape[3]
    LR = TH + 2 * _R + 6
    t = pl.program_id(1)
    row0 = t * TH

    img_scr[...] = img_ref[0, :, pl.ds(row0, LR), :]
    d = d_ref[0, 0, pl.ds(row0, LR), :]
    r = jnp.abs(d)
    r05 = r + 0.5
    bw = 1.0 / (jnp.pi * r * r + 1.0)
    di = d.astype(jnp.int32).astype(jnp.float32)
    wpl_ref[0] = bw
    cand_ref[0] = di
    for d2 in _D2S[1:]:
        m = r05 >= float(np.sqrt(d2))
        wpl_ref[_KOF[d2]] = jnp.where(m, bw, 0.0)
        cand_ref[_KOF[d2]] = jnp.where(m, di, _NEG)

    for sy in range(2 * _R + 1):
        img_sh_ref[sy] = img_scr[:, sy:sy + TH, :]

    first = True
    for adx in range(_R + 1):
        bacc = bwc = bdd = None
        for dy in range(-_YMAX[adx], _YMAX[adx] + 1):
            k = _KOF[dy * dy + adx * adx]
            sy = _R - dy
            w = wpl_ref[k, sy:sy + TH, :]
            cnd = cand_ref[k, sy:sy + TH, :]
            im = img_sh_ref[sy]
            if bacc is None:
                bacc = w[None] * im
                bwc = w
                bdd = cnd
            else:
                bacc = bacc + w[None] * im
                bwc = bwc + w
                bdd = jnp.maximum(bdd, cnd)
        b_ref[0:3] = bacc
        b_ref[3] = bwc
        b_ref[4] = bdd
        for dx in sorted({adx, -adx}):
            sx = _R - dx
            if first:
                bokeh_ref[0] = b_ref[0:3, :, sx:sx + W]
                wc_ref[...] = b_ref[3, :, sx:sx + W]
                dd_ref[0, 0] = b_ref[4, :, sx:sx + W]
                first = False
            else:
                bokeh_ref[0] = bokeh_ref[0] + b_ref[0:3, :, sx:sx + W]
                wc_ref[...] = wc_ref[...] + b_ref[3, :, sx:sx + W]
                dd_ref[0, 0] = jnp.maximum(dd_ref[0, 0],
                                           b_ref[4, :, sx:sx + W])

    inv = 1.0 / wc_ref[...]
    bokeh_ref[0] = bokeh_ref[0] * inv[None]


@jax.jit
def kernel(image, defocus):
    B, C, H, W = image.shape
    Hp = H + _R + _HALO
    Wp = W + 2 * _R
    LR = _TH + 2 * _R + 6
    K = len(_D2S)
    img_p = jnp.pad(image, ((0, 0), (0, 0), (_R, _HALO), (_R, _R)))
    d_p = jnp.pad(defocus, ((0, 0), (0, 0), (_R, _HALO), (_R, _R)))
    T = H // _TH
    bokeh, dd = pl.pallas_call(
        _bokeh_body,
        grid=(B, T),
        in_specs=[
            pl.BlockSpec((1, C, Hp, Wp), lambda b, t: (b, 0, 0, 0)),
            pl.BlockSpec((1, 1, Hp, Wp), lambda b, t: (b, 0, 0, 0)),
        ],
        out_specs=[
            pl.BlockSpec((1, C, _TH, W), lambda b, t: (b, 0, t, 0)),
            pl.BlockSpec((1, 1, _TH, W), lambda b, t: (b, 0, t, 0)),
        ],
        out_shape=[
            jax.ShapeDtypeStruct((B, C, H, W), jnp.float32),
            jax.ShapeDtypeStruct((B, 1, H, W), jnp.float32),
        ],
        scratch_shapes=[
            pltpu.VMEM((C, LR, Wp), jnp.float32),
            pltpu.VMEM((K, LR, Wp), jnp.float32),
            pltpu.VMEM((K, LR, Wp), jnp.float32),
            pltpu.VMEM((2 * _R + 1, C, _TH, Wp), jnp.float32),
            pltpu.VMEM((5, _TH, Wp), jnp.float32),
            pltpu.VMEM((_TH, W), jnp.float32),
        ],
    )(img_p, d_p)
    return bokeh, dd


# speedup vs baseline: 1.3718x; 1.3718x over previous
import numpy as np
import jax
import jax.numpy as jnp
from jax.experimental import pallas as pl
from jax.experimental.pallas import tpu as pltpu

_R = 5
_NEG = -1e9
_TH = 256
_HALO = _R + 11

_D2S = sorted({dy * dy + dx * dx
               for dy in range(-_R, _R + 1) for dx in range(-_R, _R + 1)
               if dy * dy + dx * dx < (_R + 0.5) ** 2})
_KOF = {d2: k for k, d2 in enumerate(_D2S)}
_YMAX = [int(np.floor(np.sqrt((_R + 0.5) ** 2 - 1e-9 - dx * dx)))
         for dx in range(_R + 1)]


def _bokeh_body(img_ref, d_ref, bokeh_ref, dd_ref, img_scr, wpl_ref, cand_ref,
                wc_ref):
    TH = bokeh_ref.shape[2]
    W = bokeh_ref.shape[3]
    LR = TH + 2 * _R + 6
    t = pl.program_id(1)
    row0 = t * TH

    img_scr[...] = img_ref[0, :, pl.ds(row0, LR), :]
    d = d_ref[0, 0, pl.ds(row0, LR), :]
    r = jnp.abs(d)
    r05 = r + 0.5
    bw = 1.0 / (jnp.pi * r * r + 1.0)
    di = d.astype(jnp.int32).astype(jnp.float32)
    wpl_ref[0] = bw
    cand_ref[0] = di
    for d2 in _D2S[1:]:
        m = r05 >= float(np.sqrt(d2))
        wpl_ref[_KOF[d2]] = jnp.where(m, bw, 0.0)
        cand_ref[_KOF[d2]] = jnp.where(m, di, _NEG)

    first = True
    for adx in range(_R + 1):
        bacc = bwc = bdd = None
        for dy in range(-_YMAX[adx], _YMAX[adx] + 1):
            k = _KOF[dy * dy + adx * adx]
            sy = _R - dy
            w = wpl_ref[k, sy:sy + TH, :]
            cnd = cand_ref[k, sy:sy + TH, :]
            im = img_scr[:, sy:sy + TH, :]
            if bacc is None:
                bacc = w[None] * im
                bwc = w
                bdd = cnd
            else:
                bacc = bacc + w[None] * im
                bwc = bwc + w
                bdd = jnp.maximum(bdd, cnd)
        for dx in sorted({adx, -adx}):
            sx = _R - dx
            sh = (-sx) % (W + 2 * _R)
            ba = pltpu.roll(bacc, sh, 2)[:, :, :W]
            bw2 = pltpu.roll(bwc, sh, 1)[:, :W]
            bd2 = pltpu.roll(bdd, sh, 1)[:, :W]
            if first:
                bokeh_ref[0] = ba
                wc_ref[...] = bw2
                dd_ref[0, 0] = bd2
                first = False
            else:
                bokeh_ref[0] = bokeh_ref[0] + ba
                wc_ref[...] = wc_ref[...] + bw2
                dd_ref[0, 0] = jnp.maximum(dd_ref[0, 0], bd2)

    inv = 1.0 / wc_ref[...]
    bokeh_ref[0] = bokeh_ref[0] * inv[None]


@jax.jit
def kernel(image, defocus):
    B, C, H, W = image.shape
    Hp = H + _R + _HALO
    Wp = W + 2 * _R
    LR = _TH + 2 * _R + 6
    K = len(_D2S)
    img_p = jnp.pad(image, ((0, 0), (0, 0), (_R, _HALO), (_R, _R)))
    d_p = jnp.pad(defocus, ((0, 0), (0, 0), (_R, _HALO), (_R, _R)))
    T = H // _TH
    bokeh, dd = pl.pallas_call(
        _bokeh_body,
        grid=(B, T),
        in_specs=[
            pl.BlockSpec((1, C, Hp, Wp), lambda b, t: (b, 0, 0, 0)),
            pl.BlockSpec((1, 1, Hp, Wp), lambda b, t: (b, 0, 0, 0)),
        ],
        out_specs=[
            pl.BlockSpec((1, C, _TH, W), lambda b, t: (b, 0, t, 0)),
            pl.BlockSpec((1, 1, _TH, W), lambda b, t: (b, 0, t, 0)),
        ],
        out_shape=[
            jax.ShapeDtypeStruct((B, C, H, W), jnp.float32),
            jax.ShapeDtypeStruct((B, 1, H, W), jnp.float32),
        ],
        scratch_shapes=[
            pltpu.VMEM((C, LR, Wp), jnp.float32),
            pltpu.VMEM((K, LR, Wp), jnp.float32),
            pltpu.VMEM((K, LR, Wp), jnp.float32),
            pltpu.VMEM((_TH, W), jnp.float32),
        ],
    )(img_p, d_p)
    return bokeh, dd
